# baseline (device time: 13872 ns/iter reference)
import jax
import jax.numpy as jnp
from jax import lax
from jax.experimental import pallas as pl
from jax.experimental.pallas import tpu as pltpu

T = 256
D = 512
V_LOCAL = 4096
V_HALF = 2048
CHUNK = 512
N_CHUNKS = V_HALF // CHUNK


def kernel(x, W, labels):
    def body(x_ref, w_ref, labels_ref, out_ref,
             xb_ref, acc_s, acc_ll, stats_ref, stats2_ref,
             recv_x_ref, recv_y_ref, sems):
        c = pl.program_id(0)
        my_x = lax.axis_index("x")
        my_y = lax.axis_index("y")
        nbr_x = (1 - my_x, my_y)
        nbr_y = (my_x, 1 - my_y)
        barrier_sem = pltpu.get_barrier_semaphore()

        @pl.when(c == 0)
        def _():
            for nbr in (nbr_x, nbr_y):
                pl.semaphore_signal(barrier_sem, inc=1, device_id=nbr,
                                    device_id_type=pl.DeviceIdType.MESH)
            xb_ref[:, :] = x_ref[:, :].astype(jnp.bfloat16)
            acc_s[:, :] = jnp.zeros((1, T), jnp.float32)
            acc_ll[:, :] = jnp.zeros((1, T), jnp.float32)

        wb = w_ref[:, :].astype(jnp.bfloat16)
        lgT = lax.dot_general(wb, xb_ref[:, :], (((0,), (1,)), ((), ())),
                              preferred_element_type=jnp.float32)
        acc_s[:, :] += jnp.sum(jnp.exp(lgT), axis=0, keepdims=True)
        rows = (my_x * V_HALF + c * CHUNK
                + lax.broadcasted_iota(jnp.int32, (CHUNK, T), 0))
        idx = labels_ref[:, :] - my_y * V_LOCAL
        acc_ll[:, :] += jnp.sum(jnp.where(rows == idx, lgT, 0.0),
                                axis=0, keepdims=True)

        @pl.when(c == N_CHUNKS - 1)
        def _():
            stats_ref[0:1, :] = acc_s[:, :]
            stats_ref[1:2, :] = acc_ll[:, :]
            pl.semaphore_wait(barrier_sem, 2)
            rdma_x = pltpu.make_async_remote_copy(
                src_ref=stats_ref, dst_ref=recv_x_ref,
                send_sem=sems.at[0], recv_sem=sems.at[1],
                device_id=nbr_x, device_id_type=pl.DeviceIdType.MESH,
            )
            rdma_x.start()
            rdma_x.wait()
            stats2_ref[:, :] = stats_ref[:, :] + recv_x_ref[:, :]
            rdma_y = pltpu.make_async_remote_copy(
                src_ref=stats2_ref, dst_ref=recv_y_ref,
                send_sem=sems.at[2], recv_sem=sems.at[3],
                device_id=nbr_y, device_id_type=pl.DeviceIdType.MESH,
            )
            rdma_y.start()
            rdma_y.wait()
            s = stats2_ref[0:1, :] + recv_y_ref[0:1, :]
            ll = stats2_ref[1:2, :] + recv_y_ref[1:2, :]
            out_ref[:, :] = jnp.log(s) - ll

    def w_index(c):
        return (0, lax.axis_index("x") * N_CHUNKS + c)

    out = pl.pallas_call(
        body,
        grid=(N_CHUNKS,),
        out_shape=jax.ShapeDtypeStruct((1, T), jnp.float32),
        in_specs=[
            pl.BlockSpec((T, D), lambda c: (0, 0)),
            pl.BlockSpec((D, CHUNK), w_index),
            pl.BlockSpec((1, T), lambda c: (0, 0)),
        ],
        out_specs=pl.BlockSpec((1, T), lambda c: (0, 0)),
        scratch_shapes=[
            pltpu.VMEM((T, D), jnp.bfloat16),
            pltpu.VMEM((1, T), jnp.float32),
            pltpu.VMEM((1, T), jnp.float32),
            pltpu.VMEM((2, T), jnp.float32),
            pltpu.VMEM((2, T), jnp.float32),
            pltpu.VMEM((2, T), jnp.float32),
            pltpu.VMEM((2, T), jnp.float32),
            pltpu.SemaphoreType.DMA((4,)),
        ],
        compiler_params=pltpu.CompilerParams(collective_id=0),
    )(x, W, labels.reshape(1, T))
    return out.reshape(T)


# device time: 10636 ns/iter; 1.3042x vs baseline; 1.3042x over previous
import jax
import jax.numpy as jnp
from jax import lax
from jax.experimental import pallas as pl
from jax.experimental.pallas import tpu as pltpu

T = 256
D = 512
V_LOCAL = 4096
CHUNK = 512
N_CHUNKS = V_LOCAL // CHUNK


def kernel(x, W, labels):
    def body(x_hbm, w_ref, labels_hbm, out_ref,
             x_scr, lab_scr, stats_ref, recv_ref,
             x_sem, lab_sem, send_sem, recv_sem):
        my_x = lax.axis_index("x")
        my_y = lax.axis_index("y")
        nbr = (my_x, 1 - my_y)

        x_dma = pltpu.make_async_copy(x_hbm, x_scr, x_sem)
        lab_dma = pltpu.make_async_copy(labels_hbm, lab_scr, lab_sem)
        x_dma.start()
        lab_dma.start()

        barrier_sem = pltpu.get_barrier_semaphore()
        pl.semaphore_signal(barrier_sem, inc=1, device_id=nbr,
                            device_id_type=pl.DeviceIdType.MESH)

        x_dma.wait()
        lab_dma.wait()
        xb = x_scr[:, :].astype(jnp.bfloat16)
        idx = lab_scr[:, :] - my_y * V_LOCAL
        s = jnp.zeros((1, T), jnp.float32)
        ll = jnp.zeros((1, T), jnp.float32)
        for c in range(N_CHUNKS):
            wb = w_ref[:, c * CHUNK:(c + 1) * CHUNK].astype(jnp.bfloat16)
            lgT = lax.dot_general(wb, xb, (((0,), (1,)), ((), ())),
                                  preferred_element_type=jnp.float32)
            s = s + jnp.sum(jnp.exp(lgT), axis=0, keepdims=True)
            rows = c * CHUNK + lax.broadcasted_iota(jnp.int32, (CHUNK, T), 0)
            ll = ll + jnp.sum(jnp.where(rows == idx, lgT, 0.0),
                              axis=0, keepdims=True)

        stats_ref[0:1, :] = s
        stats_ref[1:2, :] = ll
        pl.semaphore_wait(barrier_sem, 1)
        rdma = pltpu.make_async_remote_copy(
            src_ref=stats_ref, dst_ref=recv_ref,
            send_sem=send_sem, recv_sem=recv_sem,
            device_id=nbr, device_id_type=pl.DeviceIdType.MESH,
        )
        rdma.start()
        rdma.wait()
        out_ref[:, :] = (jnp.log(s + recv_ref[0:1, :])
                         - (ll + recv_ref[1:2, :]))

    out = pl.pallas_call(
        body,
        out_shape=jax.ShapeDtypeStruct((1, T), jnp.float32),
        in_specs=[
            pl.BlockSpec(memory_space=pl.ANY),
            pl.BlockSpec(memory_space=pltpu.VMEM),
            pl.BlockSpec(memory_space=pl.ANY),
        ],
        out_specs=pl.BlockSpec(memory_space=pltpu.VMEM),
        scratch_shapes=[
            pltpu.VMEM((T, D), jnp.float32),
            pltpu.VMEM((1, T), jnp.int32),
            pltpu.VMEM((2, T), jnp.float32),
            pltpu.VMEM((2, T), jnp.float32),
            pltpu.SemaphoreType.DMA,
            pltpu.SemaphoreType.DMA,
            pltpu.SemaphoreType.DMA,
            pltpu.SemaphoreType.DMA,
        ],
        compiler_params=pltpu.CompilerParams(collective_id=0),
    )(
        pltpu.with_memory_space_constraint(x, pltpu.MemorySpace.HBM),
        W,
        pltpu.with_memory_space_constraint(
            labels.reshape(1, T), pltpu.MemorySpace.HBM),
    )
    return out.reshape(T)
